# baseline (speedup 1.0000x reference)
import math

import jax
import jax.numpy as jnp
from jax.experimental import pallas as pl
from jax.experimental.pallas import tpu as pltpu

_LN125 = math.log(1.25)

_B = 8
_H = 512
_W = 512
_CHUNK_ROWS = 8
_NQ = 11


def _lane_reduce(q):
    parts = [q[:, j * 128:(j + 1) * 128] for j in range(_W // 128)]
    while len(parts) > 1:
        parts = [a + b for a, b in zip(parts[::2], parts[1::2])]
    return parts[0]


def _metrics_kernel(o_ref, t_ref, *rest):
    out_refs, acc_ref = rest[:10], rest[10]
    step = pl.program_id(0)
    nsteps = pl.num_programs(0)
    big = jnp.float32(1e30)
    one = jnp.float32(1.0)
    zero = jnp.float32(0.0)

    def body(i, carry):
        p = i // (_H // _CHUNK_ROWS)
        r = i % (_H // _CHUNK_ROWS)
        o = o_ref[p, 0, pl.ds(r * _CHUNK_ROWS, _CHUNK_ROWS), :]
        t_raw = t_ref[p, 0, pl.ds(r * _CHUNK_ROWS, _CHUNK_ROWS), :]
        m = t_raw > 0.001
        t = jnp.where(m, t_raw, o)
        mf = jnp.where(m, one, zero)

        d = jnp.abs(o - t)
        d2 = d * d
        rt = 1.0 / t
        ro = 1.0 / o
        lr = jnp.log(o * rt)
        lr2 = lr * lr
        lr2m = jnp.where(m, lr2, big)
        drt = d * rt
        rdiff = ro - rt
        qs = (
            mf,
            d2,
            d,
            lr2,
            drt,
            drt * d,
            jnp.where(lr2m < _LN125 * _LN125, one, zero),
            jnp.where(lr2m < 4.0 * _LN125 * _LN125, one, zero),
            jnp.where(lr2m < 9.0 * _LN125 * _LN125, one, zero),
            rdiff * rdiff,
            jnp.abs(rdiff),
        )
        return tuple(c + _lane_reduce(q) for c, q in zip(carry, qs))

    init = tuple(jnp.zeros((_CHUNK_ROWS, 128), jnp.float32)
                 for _ in range(_NQ))
    acc = jax.lax.fori_loop(0, 2 * _H // _CHUNK_ROWS, body, init, unroll=128)

    @pl.when(step == 0)
    def _init():
        for q in range(_NQ):
            acc_ref[q] = acc[q]

    @pl.when(step != 0)
    def _accum():
        for q in range(_NQ):
            acc_ref[q] += acc[q]

    @pl.when(step == nsteps - 1)
    def _finalize():
        s = [jnp.sum(acc_ref[q]) for q in range(_NQ)]
        inv_count = 1.0 / s[0]
        out_refs[0][0] = jnp.sqrt(s[1] * inv_count)
        out_refs[1][0] = s[2] * inv_count
        out_refs[2][0] = s[4] * inv_count
        out_refs[3][0] = s[6] * inv_count
        out_refs[4][0] = s[7] * inv_count
        out_refs[5][0] = s[8] * inv_count
        out_refs[6][0] = 1000.0 * jnp.sqrt(s[9] * inv_count)
        out_refs[7][0] = 1000.0 * s[10] * inv_count
        out_refs[8][0] = s[5] * inv_count
        out_refs[9][0] = jnp.sqrt(s[3] * inv_count)


def kernel(outputs, target):
    res = pl.pallas_call(
        _metrics_kernel,
        grid=(_B // 2,),
        in_specs=[
            pl.BlockSpec((2, 1, _H, _W), lambda i: (i, 0, 0, 0)),
            pl.BlockSpec((2, 1, _H, _W), lambda i: (i, 0, 0, 0)),
        ],
        out_specs=[pl.BlockSpec(memory_space=pltpu.SMEM)] * 10,
        out_shape=[jax.ShapeDtypeStruct((1,), jnp.float32)] * 10,
        scratch_shapes=[pltpu.VMEM((_NQ, _CHUNK_ROWS, 128), jnp.float32)],
    )(outputs, target)
    return tuple(r.reshape(()) for r in res)


# speedup vs baseline: 1.0135x; 1.0135x over previous
import math

import jax
import jax.numpy as jnp
from jax.experimental import pallas as pl
from jax.experimental.pallas import tpu as pltpu

_LN125 = math.log(1.25)

_B = 8
_H = 512
_W = 512
_CHUNK_ROWS = 16
_NQ = 11


def _lane_reduce(q):
    parts = [q[:, j * 128:(j + 1) * 128] for j in range(_W // 128)]
    while len(parts) > 1:
        parts = [a + b for a, b in zip(parts[::2], parts[1::2])]
    return parts[0]


def _metrics_kernel(o_ref, t_ref, *rest):
    out_refs, acc_ref = rest[:10], rest[10]
    step = pl.program_id(0)
    nsteps = pl.num_programs(0)
    big = jnp.float32(1e30)
    one = jnp.float32(1.0)
    zero = jnp.float32(0.0)

    def body(i, carry):
        p = i // (_H // _CHUNK_ROWS)
        r = i % (_H // _CHUNK_ROWS)
        o = o_ref[p, 0, pl.ds(r * _CHUNK_ROWS, _CHUNK_ROWS), :]
        t_raw = t_ref[p, 0, pl.ds(r * _CHUNK_ROWS, _CHUNK_ROWS), :]
        m = t_raw > 0.001
        t = jnp.where(m, t_raw, o)
        mf = jnp.where(m, one, zero)

        d = jnp.abs(o - t)
        d2 = d * d
        rt = 1.0 / t
        ro = 1.0 / o
        lr = jnp.log(o * rt)
        lr2 = lr * lr
        lr2m = jnp.where(m, lr2, big)
        drt = d * rt
        rdiff = ro - rt
        qs = (
            mf,
            d2,
            d,
            lr2,
            drt,
            drt * d,
            jnp.where(lr2m < _LN125 * _LN125, one, zero),
            jnp.where(lr2m < 4.0 * _LN125 * _LN125, one, zero),
            jnp.where(lr2m < 9.0 * _LN125 * _LN125, one, zero),
            rdiff * rdiff,
            jnp.abs(rdiff),
        )
        return tuple(c + _lane_reduce(q) for c, q in zip(carry, qs))

    init = tuple(jnp.zeros((_CHUNK_ROWS, 128), jnp.float32)
                 for _ in range(_NQ))
    acc = jax.lax.fori_loop(0, 2 * _H // _CHUNK_ROWS, body, init, unroll=64)

    @pl.when(step == 0)
    def _init():
        for q in range(_NQ):
            acc_ref[q] = acc[q]

    @pl.when(step != 0)
    def _accum():
        for q in range(_NQ):
            acc_ref[q] += acc[q]

    @pl.when(step == nsteps - 1)
    def _finalize():
        s = [jnp.sum(acc_ref[q]) for q in range(_NQ)]
        inv_count = 1.0 / s[0]
        out_refs[0][0] = jnp.sqrt(s[1] * inv_count)
        out_refs[1][0] = s[2] * inv_count
        out_refs[2][0] = s[4] * inv_count
        out_refs[3][0] = s[6] * inv_count
        out_refs[4][0] = s[7] * inv_count
        out_refs[5][0] = s[8] * inv_count
        out_refs[6][0] = 1000.0 * jnp.sqrt(s[9] * inv_count)
        out_refs[7][0] = 1000.0 * s[10] * inv_count
        out_refs[8][0] = s[5] * inv_count
        out_refs[9][0] = jnp.sqrt(s[3] * inv_count)


def kernel(outputs, target):
    res = pl.pallas_call(
        _metrics_kernel,
        grid=(_B // 2,),
        in_specs=[
            pl.BlockSpec((2, 1, _H, _W), lambda i: (i, 0, 0, 0)),
            pl.BlockSpec((2, 1, _H, _W), lambda i: (i, 0, 0, 0)),
        ],
        out_specs=[pl.BlockSpec(memory_space=pltpu.SMEM)] * 10,
        out_shape=[jax.ShapeDtypeStruct((1,), jnp.float32)] * 10,
        scratch_shapes=[pltpu.VMEM((_NQ, _CHUNK_ROWS, 128), jnp.float32)],
    )(outputs, target)
    return tuple(r.reshape(()) for r in res)
